# baseline (speedup 1.0000x reference)
import jax
import jax.numpy as jnp
from jax.experimental import pallas as pl
from jax.experimental.pallas import tpu as pltpu

_BLOCK_ROWS = 1024


def _copy_body(in_ref, out_ref):
    out_ref[...] = in_ref[...]


def kernel(sequences, batch_first, padding_value, padding_side):
    B, L, D = sequences.shape
    rows = B * L
    flat = sequences.reshape(rows, D)
    br = min(_BLOCK_ROWS, rows)
    out = pl.pallas_call(
        _copy_body,
        grid=(rows // br,),
        in_specs=[pl.BlockSpec((br, D), lambda i: (i, 0))],
        out_specs=pl.BlockSpec((br, D), lambda i: (i, 0)),
        out_shape=jax.ShapeDtypeStruct((rows, D), sequences.dtype),
    )(flat)
    return out.reshape(B, L, D)
